# baseline (speedup 1.0000x reference)
import functools

import jax
import jax.numpy as jnp
from jax import lax
from jax.experimental import pallas as pl
from jax.experimental.pallas import tpu as pltpu
from jax.experimental.pallas import tpu_sc as plsc

NE = 8
D_IN = 1024
D_HID = 512
CVLOSS_W = 0.01
SWITCHLOSS_W = 0.1
ZLOSS_W = 0.0001
N_TOK = 2048
N_ASSIGN = 2 * N_TOK
BG = 128
NB = 40
S_PAD = NB * BG

NC = 2
NS = 16
NW = NC * NS
A_PER_W = N_ASSIGN // NW
S_PER_W = S_PAD // NW
S_PER_SUB = S_PAD // NS
T_PER_W = N_TOK // NW


def _router_body(x_ref, wgt_ref, xbf_ref, ei_ref, gg_ref, loss_ref,
                 slot_ref, be_ref):
    xb = x_ref[...]
    xbf = xb.astype(jnp.bfloat16)
    xbf_ref[...] = xbf
    logits = jnp.dot(xbf, wgt_ref[...],
                     preferred_element_type=jnp.float32)
    lt = logits.T
    mx = jnp.max(lt, axis=0, keepdims=True)
    ex = jnp.exp(lt - mx)
    se = jnp.sum(ex, axis=0, keepdims=True)
    probs = ex / se
    m1 = jnp.max(probs, axis=0, keepdims=True)
    srow = jax.lax.broadcasted_iota(jnp.int32, probs.shape, 0)
    e1 = jnp.min(jnp.where(probs == m1, srow, NE), axis=0, keepdims=True)
    pwo = jnp.where(srow == e1, -jnp.inf, probs)
    m2 = jnp.max(pwo, axis=0, keepdims=True)
    e2 = jnp.min(jnp.where(pwo == m2, srow, NE), axis=0, keepdims=True)
    ei_ref[0:1, :] = e1
    ei_ref[1:2, :] = e2
    gg_ref[0:1, :] = m1
    gg_ref[1:2, :] = m2

    sel1 = srow == e1
    sel2 = srow == e2
    gsum = jnp.sum(jnp.where(sel1, m1, 0.0) + jnp.where(sel2, m2, 0.0),
                   axis=1, keepdims=True)
    cnt = jnp.sum(jnp.where(jnp.logical_and(sel1, m1 > 0), 1.0, 0.0)
                  + jnp.where(jnp.logical_and(sel2, m2 > 0), 1.0, 0.0),
                  axis=1, keepdims=True)
    psum = jnp.sum(probs, axis=1, keepdims=True)
    lse = mx + jnp.log(se)
    zsum = jnp.sum(lse * lse)

    w = gsum / jnp.maximum(jnp.sum(jnp.abs(gsum)), 1e-12)
    wm = jnp.mean(w)
    var = jnp.sum((w - wm) ** 2) / (NE - 1)
    cvloss = CVLOSS_W * var / (wm * wm + 1e-10)
    pn = psum / jnp.maximum(jnp.sum(jnp.abs(psum)), 1e-12)
    cn = cnt / jnp.maximum(jnp.sum(jnp.abs(cnt)), 1e-12)
    switchloss = SWITCHLOSS_W * (1.0 - jnp.sum(pn * cn)) * NE
    zloss = ZLOSS_W * zsum / N_TOK
    loss_ref[...] = (cvloss + switchloss + zloss).reshape(1, 1)

    oh = jnp.concatenate([jnp.where(sel1, 1.0, 0.0),
                          jnp.where(sel2, 1.0, 0.0)], axis=1)
    counts = jnp.sum(oh, axis=1, keepdims=True)
    padc = jnp.floor((counts + (BG - 1)) / BG) * BG
    ri8 = jax.lax.broadcasted_iota(jnp.int32, (NE, NE), 0)
    ci8 = jax.lax.broadcasted_iota(jnp.int32, (NE, NE), 1)
    sl8 = jnp.where(ci8 < ri8, 1.0, 0.0)
    base = jnp.dot(sl8, padc, preferred_element_type=jnp.float32)
    ri = jax.lax.broadcasted_iota(jnp.int32, (BG, BG), 0)
    ci = jax.lax.broadcasted_iota(jnp.int32, (BG, BG), 1)
    u128 = jnp.where(ri <= ci, 1.0, 0.0)
    carry = jnp.zeros((NE, 1), jnp.float32)
    for blk in range(N_ASSIGN // BG):
        ohb = oh[:, blk * BG:(blk + 1) * BG]
        p = jnp.dot(ohb, u128, preferred_element_type=jnp.float32)
        slot_blk = jnp.sum((p + carry - 1.0 + base) * ohb, axis=0,
                           keepdims=True)
        slot_ref[0:1, blk * BG:(blk + 1) * BG] = slot_blk.astype(jnp.int32)
        carry = carry + jnp.sum(ohb, axis=1, keepdims=True)
    pend = base + padc
    bi = jax.lax.broadcasted_iota(jnp.int32, (NE, 48), 1).astype(jnp.float32)
    ge = jnp.where(bi * BG >= pend, 1.0, 0.0)
    be_ref[...] = jnp.minimum(jnp.sum(ge, axis=0, keepdims=True),
                              NE - 1.0).astype(jnp.int32)


def _router(xf, wgt):
    return pl.pallas_call(
        _router_body,
        grid=(1,),
        in_specs=[
            pl.BlockSpec((N_TOK, D_IN), lambda i: (0, 0)),
            pl.BlockSpec((D_IN, NE), lambda i: (0, 0)),
        ],
        out_specs=[
            pl.BlockSpec((N_TOK, D_IN), lambda i: (0, 0)),
            pl.BlockSpec((2, N_TOK), lambda i: (0, 0)),
            pl.BlockSpec((2, N_TOK), lambda i: (0, 0)),
            pl.BlockSpec((1, 1), lambda i: (0, 0)),
            pl.BlockSpec((1, N_ASSIGN), lambda i: (0, 0)),
            pl.BlockSpec((1, 48), lambda i: (0, 0)),
        ],
        out_shape=[
            jax.ShapeDtypeStruct((N_TOK, D_IN), jnp.bfloat16),
            jax.ShapeDtypeStruct((2, N_TOK), jnp.int32),
            jax.ShapeDtypeStruct((2, N_TOK), jnp.float32),
            jax.ShapeDtypeStruct((1, 1), jnp.float32),
            jax.ShapeDtypeStruct((1, N_ASSIGN), jnp.int32),
            jax.ShapeDtypeStruct((1, 48), jnp.int32),
        ],
    )(xf, wgt.astype(jnp.bfloat16))


_SC_MESH = plsc.VectorSubcoreMesh(core_axis_name="c", subcore_axis_name="s")


def _scatter_body(slotf_ref, gflat_ref, tokp1_ref, rowp_ref, gatep_ref,
                  slotv, gv, tokv, zeroi, zerof, row_sh, gate_sh):
    c = lax.axis_index("c")
    s = lax.axis_index("s")
    w_own = c * NS + s

    pltpu.sync_copy(slotf_ref.at[pl.ds(w_own * A_PER_W, A_PER_W)], slotv)
    pltpu.sync_copy(gflat_ref.at[pl.ds(w_own * A_PER_W, A_PER_W)], gv)
    pltpu.sync_copy(tokp1_ref.at[pl.ds(w_own * A_PER_W, A_PER_W)], tokv)

    for i in range(S_PER_SUB // 16):
        zeroi[pl.ds(i * 16, 16)] = jnp.zeros((16,), jnp.int32)
        zerof[pl.ds(i * 16, 16)] = jnp.zeros((16,), jnp.float32)
    pltpu.sync_copy(zeroi, row_sh.at[pl.ds(s * S_PER_SUB, S_PER_SUB)])
    pltpu.sync_copy(zerof, gate_sh.at[pl.ds(s * S_PER_SUB, S_PER_SUB)])

    plsc.subcore_barrier()

    pltpu.sync_copy(tokv, row_sh.at[slotv], add=True)
    pltpu.sync_copy(gv, gate_sh.at[slotv], add=True)

    plsc.subcore_barrier()

    pltpu.sync_copy(row_sh.at[pl.ds(s * S_PER_SUB, S_PER_SUB)], zeroi)
    pltpu.sync_copy(zeroi,
                    rowp_ref.at[pl.ds(c * S_PAD + s * S_PER_SUB, S_PER_SUB)])
    pltpu.sync_copy(gate_sh.at[pl.ds(s * S_PER_SUB, S_PER_SUB)], zerof)
    pltpu.sync_copy(zerof,
                    gatep_ref.at[pl.ds(c * S_PAD + s * S_PER_SUB, S_PER_SUB)])


@functools.partial(
    pl.kernel,
    out_type=[
        jax.ShapeDtypeStruct((NC * S_PAD,), jnp.int32),
        jax.ShapeDtypeStruct((NC * S_PAD,), jnp.float32),
    ],
    mesh=_SC_MESH,
    scratch_types=[
        pltpu.VMEM((A_PER_W,), jnp.int32),
        pltpu.VMEM((A_PER_W,), jnp.float32),
        pltpu.VMEM((A_PER_W,), jnp.int32),
        pltpu.VMEM((S_PER_SUB,), jnp.int32),
        pltpu.VMEM((S_PER_SUB,), jnp.float32),
        pltpu.VMEM_SHARED((S_PAD,), jnp.int32),
        pltpu.VMEM_SHARED((S_PAD,), jnp.float32),
    ],
)
def _sc_scatter(slotf_ref, gflat_ref, tokp1_ref, rowp_ref, gatep_ref, *rest):
    _scatter_body(slotf_ref, gflat_ref, tokp1_ref, rowp_ref, gatep_ref,
                  *rest)


@functools.partial(
    pl.kernel,
    out_type=[
        jax.ShapeDtypeStruct((S_PAD, D_IN // 2), jnp.int32),
        jax.ShapeDtypeStruct((S_PAD,), jnp.float32),
    ],
    mesh=_SC_MESH,
    scratch_types=[
        pltpu.VMEM((S_PER_W,), jnp.int32),
        pltpu.VMEM((S_PER_W,), jnp.int32),
        pltpu.VMEM((S_PER_W,), jnp.float32),
        pltpu.VMEM((S_PER_W,), jnp.float32),
        pltpu.VMEM((S_PER_W // 2,), jnp.int32),
        pltpu.VMEM((S_PER_W // 2,), jnp.int32),
        pltpu.VMEM((S_PER_W, D_IN // 2), jnp.int32),
        pltpu.SemaphoreType.DMA,
    ],
)
def _sc_gather(rowp_ref, gatep_ref, x32_ref, xs_ref, gate_ref,
               p0, p1, q0, q1, idxa, idxb, rows, sem):
    c = lax.axis_index("c")
    s = lax.axis_index("s")
    wid = c * NS + s
    off = wid * S_PER_W
    pltpu.sync_copy(rowp_ref.at[pl.ds(off, S_PER_W)], p0)
    pltpu.sync_copy(rowp_ref.at[pl.ds(S_PAD + off, S_PER_W)], p1)
    pltpu.sync_copy(gatep_ref.at[pl.ds(off, S_PER_W)], q0)
    pltpu.sync_copy(gatep_ref.at[pl.ds(S_PAD + off, S_PER_W)], q1)
    half = S_PER_W // 2
    for i in range(half // 16):
        v = p0[pl.ds(i * 16, 16)] + p1[pl.ds(i * 16, 16)]
        idxa[pl.ds(i * 16, 16)] = jnp.maximum(v, 1) - 1
    for i in range(half // 16):
        j = half + i * 16
        v = p0[pl.ds(j, 16)] + p1[pl.ds(j, 16)]
        idxb[pl.ds(i * 16, 16)] = jnp.maximum(v, 1) - 1
    for i in range(S_PER_W // 16):
        q0[pl.ds(i * 16, 16)] = q0[pl.ds(i * 16, 16)] + q1[pl.ds(i * 16, 16)]
    pltpu.sync_copy(q0, gate_ref.at[pl.ds(off, S_PER_W)])
    pltpu.async_copy(x32_ref.at[idxa], rows.at[pl.ds(0, half)], sem).wait()
    pltpu.async_copy(x32_ref.at[idxb], rows.at[pl.ds(half, half)], sem).wait()
    pltpu.sync_copy(rows, xs_ref.at[pl.ds(off, S_PER_W)])


def _gmm_body(be_ref, xs_ref, w1_ref, w2_ref, g_ref, out_ref):
    h = jnp.maximum(jnp.dot(xs_ref[...], w1_ref[0],
                            preferred_element_type=jnp.float32), 0.0)
    o = jnp.dot(h.astype(jnp.bfloat16), w2_ref[0],
                preferred_element_type=jnp.float32)
    out_ref[...] = o * g_ref[...]


def _gmm(be, xs, w1, w2, gate):
    return pl.pallas_call(
        _gmm_body,
        grid_spec=pltpu.PrefetchScalarGridSpec(
            num_scalar_prefetch=1,
            grid=(NB,),
            in_specs=[
                pl.BlockSpec((BG, D_IN), lambda b, be: (b, 0)),
                pl.BlockSpec((1, D_IN, D_HID), lambda b, be: (be[b], 0, 0)),
                pl.BlockSpec((1, D_HID, D_IN), lambda b, be: (be[b], 0, 0)),
                pl.BlockSpec((BG, 1), lambda b, be: (b, 0)),
            ],
            out_specs=pl.BlockSpec((BG, D_IN), lambda b, be: (b, 0)),
        ),
        out_shape=jax.ShapeDtypeStruct((S_PAD, D_IN), jnp.float32),
        compiler_params=pltpu.CompilerParams(
            dimension_semantics=("arbitrary",),
        ),
    )(be, xs, w1, w2, gate)


@functools.partial(
    pl.kernel,
    out_type=jax.ShapeDtypeStruct((N_TOK, D_IN), jnp.float32),
    mesh=_SC_MESH,
    scratch_types=[
        pltpu.VMEM((2, T_PER_W // 2), jnp.int32),
        pltpu.VMEM((2, T_PER_W // 2), jnp.int32),
        pltpu.VMEM((T_PER_W // 2, D_IN), jnp.float32),
        pltpu.VMEM((T_PER_W // 2, D_IN), jnp.float32),
        pltpu.VMEM((T_PER_W // 2, D_IN), jnp.float32),
        pltpu.SemaphoreType.DMA,
    ],
)
def _sc_combine(out_ref, slot_ref, y_ref, s1h, s2h, r1, r2, yb, sem):
    c = lax.axis_index("c")
    s = lax.axis_index("s")
    wid = c * NS + s
    off = wid * T_PER_W
    half = T_PER_W // 2
    for hh in range(2):
        pltpu.sync_copy(slot_ref.at[pl.ds(off + hh * half, half)],
                        s1h.at[hh])
        pltpu.sync_copy(slot_ref.at[pl.ds(N_TOK + off + hh * half, half)],
                        s2h.at[hh])
    for hh in range(2):
        pltpu.async_copy(out_ref.at[s1h.at[hh]], r1, sem).wait()
        pltpu.async_copy(out_ref.at[s2h.at[hh]], r2, sem).wait()

        def body(t, carry):
            for cc in range(D_IN // 16):
                yb[t, pl.ds(cc * 16, 16)] = (
                    r1[t, pl.ds(cc * 16, 16)] + r2[t, pl.ds(cc * 16, 16)])
            return carry

        lax.fori_loop(0, half, body, 0)
        pltpu.sync_copy(yb, y_ref.at[pl.ds(off + hh * half, half)])


@jax.jit
def _moe_sparse(xf, wgt, w1, w2):
    xbf, ei, gg, loss, slot2d, be2d = _router(xf, wgt)
    slot = slot2d.reshape(N_ASSIGN)
    gflat = gg.reshape(N_ASSIGN)
    tokp1 = jnp.tile(jnp.arange(N_TOK, dtype=jnp.int32) + 1, 2)
    be48 = be2d.reshape(48)
    rowp, gatep = _sc_scatter(slot, gflat, tokp1)
    x32 = lax.bitcast_convert_type(xbf.reshape(N_TOK, D_IN // 2, 2),
                                   jnp.int32)
    xs32, gate = _sc_gather(rowp, gatep, x32)
    xs = lax.bitcast_convert_type(xs32, jnp.bfloat16).reshape(S_PAD, D_IN)
    out = _gmm(be48[:NB], xs,
               w1.astype(jnp.bfloat16), w2.astype(jnp.bfloat16),
               gate.reshape(S_PAD, 1))
    y = _sc_combine(out, slot)
    return y, loss


def kernel(x, Wg, W1, W2):
    bsz, length, emb = x.shape
    xf = x.reshape(-1, emb)
    y, loss = _moe_sparse(xf, Wg.T, W1, W2)
    return y.reshape(bsz, length, emb), loss[0, 0]


# speedup vs baseline: 1.7577x; 1.7577x over previous
import functools

import jax
import jax.numpy as jnp
from jax import lax
from jax.experimental import pallas as pl
from jax.experimental.pallas import tpu as pltpu
from jax.experimental.pallas import tpu_sc as plsc

NE = 8
D_IN = 1024
D_HID = 512
CVLOSS_W = 0.01
SWITCHLOSS_W = 0.1
ZLOSS_W = 0.0001
N_TOK = 2048
N_ASSIGN = 2 * N_TOK
BG = 128
NB = 40
S_PAD = NB * BG

NC = 2
NS = 16
NW = NC * NS
A_PER_W = N_ASSIGN // NW
S_PER_W = S_PAD // NW
S_PER_SUB = S_PAD // NS
T_PER_W = N_TOK // NW


def _router_body(x_ref, wgt_ref, ei_ref, gg_ref, loss_ref,
                 slot_ref, be_ref):
    xb = x_ref[...]
    xbf = xb.astype(jnp.bfloat16)
    logits = jnp.dot(xbf, wgt_ref[...],
                     preferred_element_type=jnp.float32)
    lt = logits.T
    mx = jnp.max(lt, axis=0, keepdims=True)
    ex = jnp.exp(lt - mx)
    se = jnp.sum(ex, axis=0, keepdims=True)
    probs = ex / se
    m1 = jnp.max(probs, axis=0, keepdims=True)
    srow = jax.lax.broadcasted_iota(jnp.int32, probs.shape, 0)
    e1 = jnp.min(jnp.where(probs == m1, srow, NE), axis=0, keepdims=True)
    pwo = jnp.where(srow == e1, -jnp.inf, probs)
    m2 = jnp.max(pwo, axis=0, keepdims=True)
    e2 = jnp.min(jnp.where(pwo == m2, srow, NE), axis=0, keepdims=True)
    ei_ref[0:1, :] = e1
    ei_ref[1:2, :] = e2
    gg_ref[0:1, :] = m1
    gg_ref[1:2, :] = m2

    sel1 = srow == e1
    sel2 = srow == e2
    gsum = jnp.sum(jnp.where(sel1, m1, 0.0) + jnp.where(sel2, m2, 0.0),
                   axis=1, keepdims=True)
    cnt = jnp.sum(jnp.where(jnp.logical_and(sel1, m1 > 0), 1.0, 0.0)
                  + jnp.where(jnp.logical_and(sel2, m2 > 0), 1.0, 0.0),
                  axis=1, keepdims=True)
    psum = jnp.sum(probs, axis=1, keepdims=True)
    lse = mx + jnp.log(se)
    zsum = jnp.sum(lse * lse)

    w = gsum / jnp.maximum(jnp.sum(jnp.abs(gsum)), 1e-12)
    wm = jnp.mean(w)
    var = jnp.sum((w - wm) ** 2) / (NE - 1)
    cvloss = CVLOSS_W * var / (wm * wm + 1e-10)
    pn = psum / jnp.maximum(jnp.sum(jnp.abs(psum)), 1e-12)
    cn = cnt / jnp.maximum(jnp.sum(jnp.abs(cnt)), 1e-12)
    switchloss = SWITCHLOSS_W * (1.0 - jnp.sum(pn * cn)) * NE
    zloss = ZLOSS_W * zsum / N_TOK
    loss_ref[...] = (cvloss + switchloss + zloss).reshape(1, 1)

    oh = jnp.concatenate([jnp.where(sel1, 1.0, 0.0),
                          jnp.where(sel2, 1.0, 0.0)], axis=1)
    counts = jnp.sum(oh, axis=1, keepdims=True)
    padc = jnp.floor((counts + (BG - 1)) / BG) * BG
    ri8 = jax.lax.broadcasted_iota(jnp.int32, (NE, NE), 0)
    ci8 = jax.lax.broadcasted_iota(jnp.int32, (NE, NE), 1)
    sl8 = jnp.where(ci8 < ri8, 1.0, 0.0)
    base = jnp.dot(sl8, padc, preferred_element_type=jnp.float32)
    ri = jax.lax.broadcasted_iota(jnp.int32, (BG, BG), 0)
    ci = jax.lax.broadcasted_iota(jnp.int32, (BG, BG), 1)
    u128 = jnp.where(ri <= ci, 1.0, 0.0)
    carry = jnp.zeros((NE, 1), jnp.float32)
    for blk in range(N_ASSIGN // BG):
        ohb = oh[:, blk * BG:(blk + 1) * BG]
        p = jnp.dot(ohb, u128, preferred_element_type=jnp.float32)
        slot_blk = jnp.sum((p + carry - 1.0 + base) * ohb, axis=0,
                           keepdims=True)
        slot_ref[0:1, blk * BG:(blk + 1) * BG] = slot_blk.astype(jnp.int32)
        carry = carry + jnp.sum(ohb, axis=1, keepdims=True)
    pend = base + padc
    bi = jax.lax.broadcasted_iota(jnp.int32, (NE, 48), 1).astype(jnp.float32)
    ge = jnp.where(bi * BG >= pend, 1.0, 0.0)
    be_ref[...] = jnp.minimum(jnp.sum(ge, axis=0, keepdims=True),
                              NE - 1.0).astype(jnp.int32)


def _router(xf, wgt):
    return pl.pallas_call(
        _router_body,
        grid=(1,),
        in_specs=[
            pl.BlockSpec((N_TOK, D_IN), lambda i: (0, 0)),
            pl.BlockSpec((D_IN, NE), lambda i: (0, 0)),
        ],
        out_specs=[
            pl.BlockSpec((2, N_TOK), lambda i: (0, 0)),
            pl.BlockSpec((2, N_TOK), lambda i: (0, 0)),
            pl.BlockSpec((1, 1), lambda i: (0, 0)),
            pl.BlockSpec((1, N_ASSIGN), lambda i: (0, 0)),
            pl.BlockSpec((1, 48), lambda i: (0, 0)),
        ],
        out_shape=[
            jax.ShapeDtypeStruct((2, N_TOK), jnp.int32),
            jax.ShapeDtypeStruct((2, N_TOK), jnp.float32),
            jax.ShapeDtypeStruct((1, 1), jnp.float32),
            jax.ShapeDtypeStruct((1, N_ASSIGN), jnp.int32),
            jax.ShapeDtypeStruct((1, 48), jnp.int32),
        ],
    )(xf, wgt.astype(jnp.bfloat16))


_SC_MESH = plsc.VectorSubcoreMesh(core_axis_name="c", subcore_axis_name="s")


def _scatter_body(slotf_ref, gflat_ref, tokp1_ref, rowp_ref, gatep_ref,
                  slotv, gv, tokv, zeroi, zerof, row_sh, gate_sh):
    c = lax.axis_index("c")
    s = lax.axis_index("s")
    w_own = c * NS + s

    pltpu.sync_copy(slotf_ref.at[pl.ds(w_own * A_PER_W, A_PER_W)], slotv)
    pltpu.sync_copy(gflat_ref.at[pl.ds(w_own * A_PER_W, A_PER_W)], gv)
    pltpu.sync_copy(tokp1_ref.at[pl.ds(w_own * A_PER_W, A_PER_W)], tokv)

    for i in range(S_PER_SUB // 16):
        zeroi[pl.ds(i * 16, 16)] = jnp.zeros((16,), jnp.int32)
        zerof[pl.ds(i * 16, 16)] = jnp.zeros((16,), jnp.float32)
    pltpu.sync_copy(zeroi, row_sh.at[pl.ds(s * S_PER_SUB, S_PER_SUB)])
    pltpu.sync_copy(zerof, gate_sh.at[pl.ds(s * S_PER_SUB, S_PER_SUB)])

    plsc.subcore_barrier()

    pltpu.sync_copy(tokv, row_sh.at[slotv], add=True)
    pltpu.sync_copy(gv, gate_sh.at[slotv], add=True)

    plsc.subcore_barrier()

    pltpu.sync_copy(row_sh.at[pl.ds(s * S_PER_SUB, S_PER_SUB)], zeroi)
    pltpu.sync_copy(zeroi,
                    rowp_ref.at[pl.ds(c * S_PAD + s * S_PER_SUB, S_PER_SUB)])
    pltpu.sync_copy(gate_sh.at[pl.ds(s * S_PER_SUB, S_PER_SUB)], zerof)
    pltpu.sync_copy(zerof,
                    gatep_ref.at[pl.ds(c * S_PAD + s * S_PER_SUB, S_PER_SUB)])


@functools.partial(
    pl.kernel,
    out_type=[
        jax.ShapeDtypeStruct((NC * S_PAD,), jnp.int32),
        jax.ShapeDtypeStruct((NC * S_PAD,), jnp.float32),
    ],
    mesh=_SC_MESH,
    scratch_types=[
        pltpu.VMEM((A_PER_W,), jnp.int32),
        pltpu.VMEM((A_PER_W,), jnp.float32),
        pltpu.VMEM((A_PER_W,), jnp.int32),
        pltpu.VMEM((S_PER_SUB,), jnp.int32),
        pltpu.VMEM((S_PER_SUB,), jnp.float32),
        pltpu.VMEM_SHARED((S_PAD,), jnp.int32),
        pltpu.VMEM_SHARED((S_PAD,), jnp.float32),
    ],
)
def _sc_scatter(slotf_ref, gflat_ref, tokp1_ref, rowp_ref, gatep_ref, *rest):
    _scatter_body(slotf_ref, gflat_ref, tokp1_ref, rowp_ref, gatep_ref,
                  *rest)


@functools.partial(
    pl.kernel,
    out_type=[
        jax.ShapeDtypeStruct((S_PAD, D_IN), jnp.float32),
        jax.ShapeDtypeStruct((S_PAD,), jnp.float32),
    ],
    mesh=_SC_MESH,
    scratch_types=[
        pltpu.VMEM((S_PER_W,), jnp.int32),
        pltpu.VMEM((S_PER_W,), jnp.int32),
        pltpu.VMEM((S_PER_W,), jnp.float32),
        pltpu.VMEM((S_PER_W,), jnp.float32),
        pltpu.VMEM((S_PER_W,), jnp.int32),
        pltpu.VMEM((S_PER_W // 4, D_IN), jnp.float32),
        pltpu.VMEM((S_PER_W // 4, D_IN), jnp.float32),
        pltpu.SemaphoreType.DMA,
        pltpu.SemaphoreType.DMA,
    ],
)
def _sc_gather(rowp_ref, gatep_ref, x_ref, xs_ref, gate_ref,
               p0, p1, q0, q1, idx, rowsa, rowsb, sema, semb):
    c = lax.axis_index("c")
    s = lax.axis_index("s")
    wid = c * NS + s
    off = wid * S_PER_W
    pltpu.sync_copy(rowp_ref.at[pl.ds(off, S_PER_W)], p0)
    pltpu.sync_copy(rowp_ref.at[pl.ds(S_PAD + off, S_PER_W)], p1)
    pltpu.sync_copy(gatep_ref.at[pl.ds(off, S_PER_W)], q0)
    pltpu.sync_copy(gatep_ref.at[pl.ds(S_PAD + off, S_PER_W)], q1)
    for i in range(S_PER_W // 16):
        v = p0[pl.ds(i * 16, 16)] + p1[pl.ds(i * 16, 16)]
        idx[pl.ds(i * 16, 16)] = jnp.maximum(v, 1) - 1
        q0[pl.ds(i * 16, 16)] = q0[pl.ds(i * 16, 16)] + q1[pl.ds(i * 16, 16)]
    pltpu.sync_copy(q0, gate_ref.at[pl.ds(off, S_PER_W)])
    qn = S_PER_W // 4
    bufs = (rowsa, rowsb)
    sems = (sema, semb)
    cps = [None, None, None, None]
    cps[0] = pltpu.async_copy(x_ref.at[idx.at[pl.ds(0, qn)]], rowsa, sema)
    cps[1] = pltpu.async_copy(x_ref.at[idx.at[pl.ds(qn, qn)]], rowsb, semb)
    for cch in range(4):
        cps[cch].wait()
        pltpu.sync_copy(bufs[cch % 2],
                        xs_ref.at[pl.ds(off + cch * qn, qn)])
        if cch + 2 < 4:
            cps[cch + 2] = pltpu.async_copy(
                x_ref.at[idx.at[pl.ds((cch + 2) * qn, qn)]],
                bufs[cch % 2], sems[cch % 2])


def _gmm_body(be_ref, xs_ref, w1_ref, w2_ref, g_ref, out_ref):
    h = jnp.maximum(jnp.dot(xs_ref[...].astype(jnp.bfloat16), w1_ref[0],
                            preferred_element_type=jnp.float32), 0.0)
    o = jnp.dot(h.astype(jnp.bfloat16), w2_ref[0],
                preferred_element_type=jnp.float32)
    out_ref[...] = o * g_ref[...]


def _gmm(be, xs, w1, w2, gate):
    return pl.pallas_call(
        _gmm_body,
        grid_spec=pltpu.PrefetchScalarGridSpec(
            num_scalar_prefetch=1,
            grid=(NB,),
            in_specs=[
                pl.BlockSpec((BG, D_IN), lambda b, be: (b, 0)),
                pl.BlockSpec((1, D_IN, D_HID), lambda b, be: (be[b], 0, 0)),
                pl.BlockSpec((1, D_HID, D_IN), lambda b, be: (be[b], 0, 0)),
                pl.BlockSpec((BG, 1), lambda b, be: (b, 0)),
            ],
            out_specs=pl.BlockSpec((BG, D_IN), lambda b, be: (b, 0)),
        ),
        out_shape=jax.ShapeDtypeStruct((S_PAD, D_IN), jnp.float32),
        compiler_params=pltpu.CompilerParams(
            dimension_semantics=("arbitrary",),
        ),
    )(be, xs, w1, w2, gate)


@functools.partial(
    pl.kernel,
    out_type=jax.ShapeDtypeStruct((N_TOK, D_IN), jnp.float32),
    mesh=_SC_MESH,
    scratch_types=[
        pltpu.VMEM((2, T_PER_W // 2), jnp.int32),
        pltpu.VMEM((2, T_PER_W // 2), jnp.int32),
        pltpu.VMEM((T_PER_W // 2, D_IN), jnp.float32),
        pltpu.VMEM((T_PER_W // 2, D_IN), jnp.float32),
        pltpu.VMEM((T_PER_W // 2, D_IN), jnp.float32),
        pltpu.SemaphoreType.DMA,
        pltpu.SemaphoreType.DMA,
    ],
)
def _sc_combine(out_ref, slot_ref, y_ref, s1h, s2h, r1, r2, yb, sem, sem2):
    c = lax.axis_index("c")
    s = lax.axis_index("s")
    wid = c * NS + s
    off = wid * T_PER_W
    half = T_PER_W // 2
    for hh in range(2):
        pltpu.sync_copy(slot_ref.at[pl.ds(off + hh * half, half)],
                        s1h.at[hh])
        pltpu.sync_copy(slot_ref.at[pl.ds(N_TOK + off + hh * half, half)],
                        s2h.at[hh])
    for hh in range(2):
        cp1 = pltpu.async_copy(out_ref.at[s1h.at[hh]], r1, sem)
        cp2 = pltpu.async_copy(out_ref.at[s2h.at[hh]], r2, sem2)
        cp1.wait()
        cp2.wait()

        def body(t, carry):
            for cc in range(D_IN // 16):
                yb[t, pl.ds(cc * 16, 16)] = (
                    r1[t, pl.ds(cc * 16, 16)] + r2[t, pl.ds(cc * 16, 16)])
            return carry

        lax.fori_loop(0, half, body, 0)
        pltpu.sync_copy(yb, y_ref.at[pl.ds(off + hh * half, half)])


@jax.jit
def _moe_sparse(xf, wgt, w1, w2):
    ei, gg, loss, slot2d, be2d = _router(xf, wgt)
    slot = slot2d.reshape(N_ASSIGN)
    gflat = gg.reshape(N_ASSIGN)
    tokp1 = jnp.tile(jnp.arange(N_TOK, dtype=jnp.int32) + 1, 2)
    be48 = be2d.reshape(48)
    rowp, gatep = _sc_scatter(slot, gflat, tokp1)
    xs, gate = _sc_gather(rowp, gatep, xf)
    out = _gmm(be48[:NB], xs,
               w1.astype(jnp.bfloat16), w2.astype(jnp.bfloat16),
               gate.reshape(S_PAD, 1))
    y = _sc_combine(out, slot)
    return y, loss


def kernel(x, Wg, W1, W2):
    bsz, length, emb = x.shape
    xf = x.reshape(-1, emb)
    y, loss = _moe_sparse(xf, Wg.T, W1, W2)
    return y.reshape(bsz, length, emb), loss[0, 0]


# speedup vs baseline: 4.6706x; 2.6572x over previous
import jax
import jax.numpy as jnp
from jax.experimental import pallas as pl
from jax.experimental.pallas import tpu as pltpu

NE = 8
D_IN = 1024
D_HID = 512
CVLOSS_W = 0.01
SWITCHLOSS_W = 0.1
ZLOSS_W = 0.0001
BT = 512
N_TOK = 2048


def _moe_body(x_ref, wgt_ref, w1_ref, w2_ref, y_ref, loss_ref,
              gsum_ref, psum_ref, cnt_ref, zsum_ref):
    tb = pl.program_id(0)
    nt = pl.num_programs(0)

    @pl.when(tb == 0)
    def _init():
        gsum_ref[...] = jnp.zeros_like(gsum_ref)
        psum_ref[...] = jnp.zeros_like(psum_ref)
        cnt_ref[...] = jnp.zeros_like(cnt_ref)
        zsum_ref[...] = jnp.zeros_like(zsum_ref)

    xb = x_ref[...]
    logits = jnp.dot(xb, wgt_ref[...],
                     preferred_element_type=jnp.float32)
    mx = jnp.max(logits, axis=1, keepdims=True)
    ex = jnp.exp(logits - mx)
    se = jnp.sum(ex, axis=1, keepdims=True)
    probs = ex / se
    m1 = jnp.max(probs, axis=1, keepdims=True)
    lane = jax.lax.broadcasted_iota(jnp.int32, probs.shape, 1)
    first_m1 = jnp.min(jnp.where(probs == m1, lane, NE), axis=1,
                       keepdims=True)
    probs_wo = jnp.where(lane == first_m1, -jnp.inf, probs)
    m2 = jnp.max(probs_wo, axis=1, keepdims=True)
    keep = probs >= m2
    gates = jnp.where(keep, probs, 0.0)
    gsum_ref[...] += jnp.sum(gates, axis=0, keepdims=True)
    psum_ref[...] += jnp.sum(probs, axis=0, keepdims=True)
    cnt_ref[...] += jnp.sum(
        jnp.where(gates > 0, 1.0, 0.0), axis=0, keepdims=True)
    lse = mx[:, 0] + jnp.log(se[:, 0])
    zsum_ref[...] += jnp.sum(lse * lse).reshape(1, 1)

    acc = jnp.zeros((BT, D_IN), jnp.float32)
    for e in range(NE):
        h = jnp.maximum(jnp.dot(xb, w1_ref[e],
                                preferred_element_type=jnp.float32), 0.0)
        o = jnp.dot(h.astype(jnp.bfloat16), w2_ref[e],
                    preferred_element_type=jnp.float32)
        acc = acc + gates[:, e:e + 1] * o
    y_ref[...] = acc

    @pl.when(tb == nt - 1)
    def _loss():
        gs = gsum_ref[...]
        w = gs / jnp.maximum(jnp.sum(jnp.abs(gs)), 1e-12)
        wm = jnp.mean(w)
        var = jnp.sum((w - wm) ** 2) / (NE - 1)
        cvloss = CVLOSS_W * var / (wm * wm + 1e-10)
        pn = psum_ref[...]
        pn = pn / jnp.maximum(jnp.sum(jnp.abs(pn)), 1e-12)
        cn = cnt_ref[...]
        cn = cn / jnp.maximum(jnp.sum(jnp.abs(cn)), 1e-12)
        switchloss = SWITCHLOSS_W * (1.0 - jnp.sum(pn * cn)) * NE
        zloss = ZLOSS_W * jnp.sum(zsum_ref[...]) / N_TOK
        loss_ref[...] = (cvloss + switchloss + zloss).reshape(1, 1)


@jax.jit
def _moe_fused(xf, wgt, w1, w2):
    nt = N_TOK // BT
    y, loss = pl.pallas_call(
        _moe_body,
        grid=(nt,),
        in_specs=[
            pl.BlockSpec((BT, D_IN), lambda tb: (tb, 0)),
            pl.BlockSpec((D_IN, NE), lambda tb: (0, 0)),
            pl.BlockSpec((NE, D_IN, D_HID), lambda tb: (0, 0, 0)),
            pl.BlockSpec((NE, D_HID, D_IN), lambda tb: (0, 0, 0)),
        ],
        out_specs=[
            pl.BlockSpec((BT, D_IN), lambda tb: (tb, 0)),
            pl.BlockSpec((1, 1), lambda tb: (0, 0)),
        ],
        out_shape=[
            jax.ShapeDtypeStruct((N_TOK, D_IN), jnp.float32),
            jax.ShapeDtypeStruct((1, 1), jnp.float32),
        ],
        scratch_shapes=[
            pltpu.VMEM((1, NE), jnp.float32),
            pltpu.VMEM((1, NE), jnp.float32),
            pltpu.VMEM((1, NE), jnp.float32),
            pltpu.VMEM((1, 1), jnp.float32),
        ],
        compiler_params=pltpu.CompilerParams(
            dimension_semantics=("arbitrary",),
        ),
    )(xf.astype(jnp.bfloat16), wgt.astype(jnp.bfloat16),
      w1.astype(jnp.bfloat16), w2.astype(jnp.bfloat16))
    return y, loss


def kernel(x, Wg, W1, W2):
    bsz, length, emb = x.shape
    xf = x.reshape(-1, emb)
    y, loss = _moe_fused(xf, Wg.T, W1, W2)
    return y.reshape(bsz, length, emb), loss[0, 0]
